# baseline (speedup 1.0000x reference)
import functools

import jax
import jax.numpy as jnp
from jax import lax
from jax.experimental import pallas as pl
from jax.experimental.pallas import tpu as pltpu
from jax.experimental.pallas import tpu_sc as plsc

BATCH, NBOX, C = 2, 512, 256
R = BATCH * NBOX
CROP = 7
PIX = R * CROP * CROP
K = CROP * CROP * 4
KP = 208
HALF = KP // 2
PXA = HALF // 4
PXB = CROP * CROP - PXA

NC, NS = 2, 16
NW = NC * NS
ROIS_PER_W = R // NW

LVL_SIZES = (256, 128, 64, 32, 16)
_LVL_BASE = []
_acc = 0
for _li, _s in enumerate(LVL_SIZES):
    _LVL_BASE.append(_acc)
    _acc += (1 if _li == 0 else BATCH) * _s * _s
TABLE_ROWS = _acc

_GATHER_DNUMS = lax.GatherDimensionNumbers(
    offset_dims=(), collapsed_slice_dims=(0,), start_index_map=(0,))


def _index_weight_body(bb_ref, idx_ref, w_ref):
    x1 = jnp.clip(bb_ref[0], 0.0, 1024.0) * (1.0 / 1024.0)
    y1 = jnp.clip(bb_ref[1], 0.0, 1024.0) * (1.0 / 1024.0)
    x2 = jnp.clip(bb_ref[2], 0.0, 1024.0) * (1.0 / 1024.0)
    y2 = jnp.clip(bb_ref[3], 0.0, 1024.0) * (1.0 / 1024.0)
    h = y2 - y1
    w = x2 - x1
    lvl_f = jnp.log(jnp.sqrt(h * w) / (56.0 / 1024.0)) / jnp.log(2.0)
    lvl = jnp.minimum(4, jnp.maximum(0, jnp.round(lvl_f).astype(jnp.int32)))

    wl = jnp.right_shift(jnp.full_like(lvl, 256), lvl)
    hm1f = wl.astype(jnp.float32) - 1.0
    hm1i = wl - 1
    base = jnp.full_like(lvl, _LVL_BASE[4])
    for l in range(3, -1, -1):
        base = jnp.where(lvl == l, _LVL_BASE[l], base)
    row_id = lax.broadcasted_iota(jnp.int32, (8, 128), 0)
    bidx = (row_id >= 4).astype(jnp.int32)
    bidx = jnp.where(lvl == 0, 0, bidx)
    roi_base = base + bidx * wl * wl

    ys_parts = []
    xs_parts = []
    for i in range(CROP):
        ys = y1 * hm1f + i * (h * hm1f) * (1.0 / 6.0)
        xs = x1 * hm1f + i * (w * hm1f) * (1.0 / 6.0)
        y0f = jnp.floor(ys)
        x0f = jnp.floor(xs)
        ly = ys - y0f
        lx = xs - x0f
        y0 = jnp.clip(y0f.astype(jnp.int32), 0, hm1i)
        yp = jnp.clip(y0f.astype(jnp.int32) + 1, 0, hm1i)
        x0 = jnp.clip(x0f.astype(jnp.int32), 0, hm1i)
        xp = jnp.clip(x0f.astype(jnp.int32) + 1, 0, hm1i)
        ys_parts.append((y0, yp, ly))
        xs_parts.append((x0, xp, lx))

    for i in range(CROP):
        y0, yp, ly = ys_parts[i]
        ry0 = roi_base + y0 * wl
        ry1 = roi_base + yp * wl
        omly = 1.0 - ly
        for j in range(CROP):
            x0, xp, lx = xs_parts[j]
            k = (i * CROP + j) * 4
            idx_ref[k + 0] = ry0 + x0
            idx_ref[k + 1] = ry0 + xp
            idx_ref[k + 2] = ry1 + x0
            idx_ref[k + 3] = ry1 + xp
            w_ref[k + 0] = omly * (1.0 - lx)
            w_ref[k + 1] = omly * lx
            w_ref[k + 2] = ly * (1.0 - lx)
            w_ref[k + 3] = ly * lx
    izero = jnp.zeros((8, 128), jnp.int32)
    fzero = jnp.zeros((8, 128), jnp.float32)
    for k in range(K, KP):
        idx_ref[k] = izero
        w_ref[k] = fzero


def _index_weights(bb_t):
    return pl.pallas_call(
        _index_weight_body,
        out_shape=[
            jax.ShapeDtypeStruct((KP, 8, 128), jnp.int32),
            jax.ShapeDtypeStruct((KP, 8, 128), jnp.float32),
        ],
    )(bb_t)


@functools.cache
def _sc_gather_blend():
    mesh = plsc.VectorSubcoreMesh(
        core_axis_name="c", subcore_axis_name="s",
        num_cores=NC, num_subcores=NS,
    )

    @functools.partial(
        pl.kernel,
        out_type=jax.ShapeDtypeStruct((BATCH, NBOX, CROP, CROP, C),
                                      jnp.float32),
        mesh=mesh,
        scratch_types=[
            pltpu.VMEM((ROIS_PER_W * KP,), jnp.int32),
            pltpu.VMEM((ROIS_PER_W * KP // 16, 16), jnp.float32),
            pltpu.VMEM((HALF, C), jnp.float32),
            pltpu.VMEM((HALF, C), jnp.float32),
            pltpu.VMEM((CROP, CROP, C), jnp.float32),
            pltpu.SemaphoreType.DMA,
            pltpu.SemaphoreType.DMA,
            pltpu.SemaphoreType.DMA,
        ],
    )
    def sc_body(idx_hbm, w_hbm, table_hbm, out_hbm,
                idx_v, w_v, rows0, rows1, out_v,
                sem0, sem1, osem):
        wid = lax.axis_index("s") * NC + lax.axis_index("c")
        rbase = wid * ROIS_PER_W

        pltpu.sync_copy(
            idx_hbm.at[pl.ds(wid * ROIS_PER_W * KP, ROIS_PER_W * KP)], idx_v)
        pltpu.sync_copy(
            w_hbm.at[pl.ds(wid * (ROIS_PER_W * KP // 16),
                           ROIS_PER_W * KP // 16)], w_v)

        def fire(i, half, rowsv, sem):
            off = i * KP + half * HALF
            pltpu.async_copy(
                table_hbm.at[idx_v.at[pl.ds(off, HALF)]], rowsv, sem)

        def drain_rows(rowsv, sem):
            pltpu.make_async_copy(
                table_hbm.at[pl.ds(0, HALF)], rowsv, sem).wait()

        def put(i, outv, osem):
            r = rbase + i
            pltpu.async_copy(outv, out_hbm.at[r // NBOX, r % NBOX], osem)

        def drain_out(outv, osem):
            pltpu.make_async_copy(outv, out_hbm.at[0, 0], osem).wait()

        def blend(i, half, rowsv, outv):
            npx = PXB if half else PXA

            def pix_body(q, c2):
                qg = half * PXA + q
                p = i * KP + half * HALF + 4 * q
                w16 = w_v[p // 16, :]
                lb = p % 16

                def splat(t):
                    iv = jnp.full((16,), lb + t, jnp.int32)
                    return lax.gather(
                        w16, iv[:, None], _GATHER_DNUMS, (1,),
                        mode=lax.GatherScatterMode.PROMISE_IN_BOUNDS)

                w0 = splat(0)
                w1 = splat(1)
                w2 = splat(2)
                w3 = splat(3)
                i7 = qg // CROP
                j7 = qg % CROP
                for cc in range(C // 16):
                    s = pl.ds(cc * 16, 16)
                    acc = (w0 * rowsv[4 * q, s] + w1 * rowsv[4 * q + 1, s]
                           + w2 * rowsv[4 * q + 2, s]
                           + w3 * rowsv[4 * q + 3, s])
                    outv[i7, j7, s] = acc
                return c2

            lax.fori_loop(0, npx, pix_body, 0)

        fire(0, 0, rows0, sem0)
        fire(0, 1, rows1, sem1)

        def body(i, carry):
            drain_rows(rows0, sem0)

            @pl.when(i > 0)
            def _():
                drain_out(out_v, osem)

            blend(i, 0, rows0, out_v)

            @pl.when(i + 1 < ROIS_PER_W)
            def _():
                fire(i + 1, 0, rows0, sem0)

            drain_rows(rows1, sem1)
            blend(i, 1, rows1, out_v)

            @pl.when(i + 1 < ROIS_PER_W)
            def _():
                fire(i + 1, 1, rows1, sem1)

            put(i, out_v, osem)
            return carry

        lax.fori_loop(0, ROIS_PER_W, body, 0)
        drain_out(out_v, osem)

    return sc_body


def kernel(bboxes, fm0, fm1, fm2, fm3, fm4):
    bb_t = bboxes.reshape(R, 4).T.reshape(4, 8, 128)
    idx_t, w_t = _index_weights(bb_t)
    idx = idx_t.reshape(KP, R).T.reshape(-1)
    wts = w_t.reshape(KP, R).T.reshape(-1, 16)
    table = jnp.concatenate(
        [fm0[0].reshape(-1, C)]
        + [fm.reshape(-1, C) for fm in (fm1, fm2, fm3, fm4)], axis=0)
    return _sc_gather_blend()(idx, wts, table)


# speedup vs baseline: 1.7496x; 1.7496x over previous
import functools

import jax
import jax.numpy as jnp
from jax import lax
from jax.experimental import pallas as pl
from jax.experimental.pallas import tpu as pltpu
from jax.experimental.pallas import tpu_sc as plsc

BATCH, NBOX, C = 2, 512, 256
R = BATCH * NBOX
CROP = 7
PIX = R * CROP * CROP
K = CROP * CROP * 4

NC, NS = 2, 16
NW = NC * NS
CHUNK = 32
CHUNKS_PER_W = PIX // (NW * CHUNK)
TAPS_PER_W = PIX // NW * 4

LVL_SIZES = (256, 128, 64, 32, 16)
_LVL_BASE = []
_acc = 0
for _s in LVL_SIZES:
    _LVL_BASE.append(_acc)
    _acc += BATCH * _s * _s
TABLE_ROWS = _acc

_GATHER_DNUMS = lax.GatherDimensionNumbers(
    offset_dims=(), collapsed_slice_dims=(0,), start_index_map=(0,))


def _index_weight_body(bb_ref, idx_ref, w_ref):
    x1 = jnp.clip(bb_ref[0], 0.0, 1024.0) * (1.0 / 1024.0)
    y1 = jnp.clip(bb_ref[1], 0.0, 1024.0) * (1.0 / 1024.0)
    x2 = jnp.clip(bb_ref[2], 0.0, 1024.0) * (1.0 / 1024.0)
    y2 = jnp.clip(bb_ref[3], 0.0, 1024.0) * (1.0 / 1024.0)
    h = y2 - y1
    w = x2 - x1
    lvl_f = jnp.log(jnp.sqrt(h * w) / (56.0 / 1024.0)) / jnp.log(2.0)
    lvl = jnp.minimum(4, jnp.maximum(0, jnp.round(lvl_f).astype(jnp.int32)))

    wl = jnp.right_shift(jnp.full_like(lvl, 256), lvl)
    hm1f = wl.astype(jnp.float32) - 1.0
    hm1i = wl - 1
    base = jnp.full_like(lvl, _LVL_BASE[4])
    for l in range(3, -1, -1):
        base = jnp.where(lvl == l, _LVL_BASE[l], base)
    row_id = lax.broadcasted_iota(jnp.int32, (8, 128), 0)
    bidx = (row_id >= 4).astype(jnp.int32)
    bidx = jnp.where(lvl == 0, 0, bidx)
    roi_base = base + bidx * wl * wl

    ys_parts = []
    xs_parts = []
    for i in range(CROP):
        ys = y1 * hm1f + i * (h * hm1f) * (1.0 / 6.0)
        xs = x1 * hm1f + i * (w * hm1f) * (1.0 / 6.0)
        y0f = jnp.floor(ys)
        x0f = jnp.floor(xs)
        ly = ys - y0f
        lx = xs - x0f
        y0 = jnp.clip(y0f.astype(jnp.int32), 0, hm1i)
        yp = jnp.clip(y0f.astype(jnp.int32) + 1, 0, hm1i)
        x0 = jnp.clip(x0f.astype(jnp.int32), 0, hm1i)
        xp = jnp.clip(x0f.astype(jnp.int32) + 1, 0, hm1i)
        ys_parts.append((y0, yp, ly))
        xs_parts.append((x0, xp, lx))

    for i in range(CROP):
        y0, yp, ly = ys_parts[i]
        ry0 = roi_base + y0 * wl
        ry1 = roi_base + yp * wl
        omly = 1.0 - ly
        for j in range(CROP):
            x0, xp, lx = xs_parts[j]
            k = (i * CROP + j) * 4
            idx_ref[k + 0] = ry0 + x0
            idx_ref[k + 1] = ry0 + xp
            idx_ref[k + 2] = ry1 + x0
            idx_ref[k + 3] = ry1 + xp
            w_ref[k + 0] = omly * (1.0 - lx)
            w_ref[k + 1] = omly * lx
            w_ref[k + 2] = ly * (1.0 - lx)
            w_ref[k + 3] = ly * lx


def _index_weights(bb_t):
    return pl.pallas_call(
        _index_weight_body,
        out_shape=[
            jax.ShapeDtypeStruct((K, 8, 128), jnp.int32),
            jax.ShapeDtypeStruct((K, 8, 128), jnp.float32),
        ],
    )(bb_t)


@functools.cache
def _sc_gather_blend():
    mesh = plsc.VectorSubcoreMesh(
        core_axis_name="c", subcore_axis_name="s",
        num_cores=NC, num_subcores=NS,
    )

    @functools.partial(
        pl.kernel,
        out_type=jax.ShapeDtypeStruct((PIX, C), jnp.float32),
        mesh=mesh,
        scratch_types=[
            pltpu.VMEM((TAPS_PER_W,), jnp.int32),
            pltpu.VMEM((TAPS_PER_W // 16, 16), jnp.float32),
            pltpu.VMEM((4 * CHUNK, C), jnp.float32),
            pltpu.VMEM((4 * CHUNK, C), jnp.float32),
            pltpu.VMEM((CHUNK, C), jnp.float32),
            pltpu.SemaphoreType.DMA,
            pltpu.SemaphoreType.DMA,
            pltpu.SemaphoreType.DMA,
        ],
    )
    def sc_body(idx_hbm, w_hbm, table_hbm, out_hbm,
                idx_v, w_v, rows0, rows1, out_v,
                sem0, sem1, osem):
        wid = lax.axis_index("s") * NC + lax.axis_index("c")
        base = wid * CHUNKS_PER_W

        pltpu.sync_copy(idx_hbm.at[pl.ds(wid * TAPS_PER_W, TAPS_PER_W)], idx_v)
        pltpu.sync_copy(w_hbm.at[pl.ds(wid * (TAPS_PER_W // 16),
                                       TAPS_PER_W // 16)], w_v)

        def fire(it, rowsv, sem):
            pltpu.async_copy(
                table_hbm.at[idx_v.at[pl.ds(it * 4 * CHUNK, 4 * CHUNK)]],
                rowsv, sem)

        def drain_rows(rowsv, sem):
            pltpu.make_async_copy(
                table_hbm.at[pl.ds(0, 4 * CHUNK)], rowsv, sem).wait()

        def put(it, outv, osem):
            pix = (base + it) * CHUNK
            pltpu.async_copy(outv, out_hbm.at[pl.ds(pix, CHUNK)], osem)

        def drain_out(outv, osem):
            pltpu.make_async_copy(
                outv, out_hbm.at[pl.ds(0, CHUNK)], osem).wait()

        def blend(it, rowsv, outv):
            def pix_body(q, c2):
                qg = it * CHUNK + q
                w16 = w_v[qg // 4, :]
                lb = (qg % 4) * 4

                def splat(t):
                    iv = jnp.full((16,), lb + t, jnp.int32)
                    return lax.gather(
                        w16, iv[:, None], _GATHER_DNUMS, (1,),
                        mode=lax.GatherScatterMode.PROMISE_IN_BOUNDS)

                w0 = splat(0)
                w1 = splat(1)
                w2 = splat(2)
                w3 = splat(3)
                for cc in range(C // 16):
                    s = pl.ds(cc * 16, 16)
                    acc = (w0 * rowsv[4 * q, s] + w1 * rowsv[4 * q + 1, s]
                           + w2 * rowsv[4 * q + 2, s]
                           + w3 * rowsv[4 * q + 3, s])
                    outv[q, s] = acc
                return c2

            lax.fori_loop(0, CHUNK, pix_body, 0)

        fire(0, rows0, sem0)
        fire(1, rows1, sem1)

        def body(g, carry):
            it0 = 2 * g
            it1 = 2 * g + 1
            drain_rows(rows0, sem0)

            @pl.when(g > 0)
            def _():
                drain_out(out_v, osem)

            blend(it0, rows0, out_v)

            @pl.when(it0 + 2 < CHUNKS_PER_W)
            def _():
                fire(it0 + 2, rows0, sem0)

            put(it0, out_v, osem)

            drain_rows(rows1, sem1)
            drain_out(out_v, osem)
            blend(it1, rows1, out_v)

            @pl.when(it1 + 2 < CHUNKS_PER_W)
            def _():
                fire(it1 + 2, rows1, sem1)

            put(it1, out_v, osem)
            return carry

        lax.fori_loop(0, CHUNKS_PER_W // 2, body, 0)
        drain_rows(rows0, sem0)
        drain_out(out_v, osem)
        blend(CHUNKS_PER_W - 1, rows0, out_v)
        put(CHUNKS_PER_W - 1, out_v, osem)
        drain_out(out_v, osem)

    return sc_body


def kernel(bboxes, fm0, fm1, fm2, fm3, fm4):
    bb_t = bboxes.reshape(R, 4).T.reshape(4, 8, 128)
    idx_t, w_t = _index_weights(bb_t)
    idx = idx_t.reshape(K, R).T.reshape(-1)
    wts = w_t.reshape(K, R).T.reshape(-1, 16)
    table = jnp.concatenate(
        [fm.reshape(-1, C) for fm in (fm0, fm1, fm2, fm3, fm4)], axis=0)
    out = _sc_gather_blend()(idx, wts, table)
    return out.reshape(BATCH, NBOX, CROP, CROP, C)


# speedup vs baseline: 1.8159x; 1.0379x over previous
import functools

import jax
import jax.numpy as jnp
from jax import lax
from jax.experimental import pallas as pl
from jax.experimental.pallas import tpu as pltpu
from jax.experimental.pallas import tpu_sc as plsc

BATCH, NBOX, C = 2, 512, 256
R = BATCH * NBOX
CROP = 7
PIX = R * CROP * CROP
K = CROP * CROP * 4

NC, NS = 2, 16
NW = NC * NS
CHUNK = 16
CHUNKS_PER_W = PIX // (NW * CHUNK)
TAPS_PER_W = PIX // NW * 4

LVL_SIZES = (256, 128, 64, 32, 16)
_LVL_BASE = []
_acc = 0
for _s in LVL_SIZES:
    _LVL_BASE.append(_acc)
    _acc += BATCH * _s * _s
TABLE_ROWS = _acc

_GATHER_DNUMS = lax.GatherDimensionNumbers(
    offset_dims=(), collapsed_slice_dims=(0,), start_index_map=(0,))


def _index_weight_body(bb_ref, idx_ref, w_ref):
    x1 = jnp.clip(bb_ref[0], 0.0, 1024.0) * (1.0 / 1024.0)
    y1 = jnp.clip(bb_ref[1], 0.0, 1024.0) * (1.0 / 1024.0)
    x2 = jnp.clip(bb_ref[2], 0.0, 1024.0) * (1.0 / 1024.0)
    y2 = jnp.clip(bb_ref[3], 0.0, 1024.0) * (1.0 / 1024.0)
    h = y2 - y1
    w = x2 - x1
    lvl_f = jnp.log(jnp.sqrt(h * w) / (56.0 / 1024.0)) / jnp.log(2.0)
    lvl = jnp.minimum(4, jnp.maximum(0, jnp.round(lvl_f).astype(jnp.int32)))

    wl = jnp.right_shift(jnp.full_like(lvl, 256), lvl)
    hm1f = wl.astype(jnp.float32) - 1.0
    hm1i = wl - 1
    base = jnp.full_like(lvl, _LVL_BASE[4])
    for l in range(3, -1, -1):
        base = jnp.where(lvl == l, _LVL_BASE[l], base)
    row_id = lax.broadcasted_iota(jnp.int32, (8, 128), 0)
    bidx = (row_id >= 4).astype(jnp.int32)
    bidx = jnp.where(lvl == 0, 0, bidx)
    roi_base = base + bidx * wl * wl

    ys_parts = []
    xs_parts = []
    for i in range(CROP):
        ys = y1 * hm1f + i * (h * hm1f) * (1.0 / 6.0)
        xs = x1 * hm1f + i * (w * hm1f) * (1.0 / 6.0)
        y0f = jnp.floor(ys)
        x0f = jnp.floor(xs)
        ly = ys - y0f
        lx = xs - x0f
        y0 = jnp.clip(y0f.astype(jnp.int32), 0, hm1i)
        yp = jnp.clip(y0f.astype(jnp.int32) + 1, 0, hm1i)
        x0 = jnp.clip(x0f.astype(jnp.int32), 0, hm1i)
        xp = jnp.clip(x0f.astype(jnp.int32) + 1, 0, hm1i)
        ys_parts.append((y0, yp, ly))
        xs_parts.append((x0, xp, lx))

    for i in range(CROP):
        y0, yp, ly = ys_parts[i]
        ry0 = roi_base + y0 * wl
        ry1 = roi_base + yp * wl
        omly = 1.0 - ly
        for j in range(CROP):
            x0, xp, lx = xs_parts[j]
            k = (i * CROP + j) * 4
            idx_ref[k + 0] = ry0 + x0
            idx_ref[k + 1] = ry0 + xp
            idx_ref[k + 2] = ry1 + x0
            idx_ref[k + 3] = ry1 + xp
            w_ref[k + 0] = omly * (1.0 - lx)
            w_ref[k + 1] = omly * lx
            w_ref[k + 2] = ly * (1.0 - lx)
            w_ref[k + 3] = ly * lx


def _index_weights(bb_t):
    return pl.pallas_call(
        _index_weight_body,
        out_shape=[
            jax.ShapeDtypeStruct((K, 8, 128), jnp.int32),
            jax.ShapeDtypeStruct((K, 8, 128), jnp.float32),
        ],
    )(bb_t)


@functools.cache
def _sc_gather_blend():
    mesh = plsc.VectorSubcoreMesh(
        core_axis_name="c", subcore_axis_name="s",
        num_cores=NC, num_subcores=NS,
    )

    @functools.partial(
        pl.kernel,
        out_type=jax.ShapeDtypeStruct((PIX, C), jnp.float32),
        mesh=mesh,
        scratch_types=[
            pltpu.VMEM((TAPS_PER_W,), jnp.int32),
            pltpu.VMEM((TAPS_PER_W // 16, 16), jnp.float32),
            pltpu.VMEM((4 * CHUNK, C), jnp.float32),
            pltpu.VMEM((4 * CHUNK, C), jnp.float32),
            pltpu.VMEM((CHUNK, C), jnp.float32),
            pltpu.VMEM((CHUNK, C), jnp.float32),
            pltpu.SemaphoreType.DMA,
            pltpu.SemaphoreType.DMA,
            pltpu.SemaphoreType.DMA,
            pltpu.SemaphoreType.DMA,
        ],
    )
    def sc_body(idx_hbm, w_hbm, table_hbm, out_hbm,
                idx_v, w_v, rows0, rows1, out0, out1,
                sem0, sem1, osem0, osem1):
        wid = lax.axis_index("s") * NC + lax.axis_index("c")
        base = wid * CHUNKS_PER_W

        pltpu.sync_copy(idx_hbm.at[pl.ds(wid * TAPS_PER_W, TAPS_PER_W)], idx_v)
        pltpu.sync_copy(w_hbm.at[pl.ds(wid * (TAPS_PER_W // 16),
                                       TAPS_PER_W // 16)], w_v)

        def fire(it, rowsv, sem):
            pltpu.async_copy(
                table_hbm.at[idx_v.at[pl.ds(it * 4 * CHUNK, 4 * CHUNK)]],
                rowsv, sem)

        def drain_rows(rowsv, sem):
            pltpu.make_async_copy(
                table_hbm.at[pl.ds(0, 4 * CHUNK)], rowsv, sem).wait()

        def put(it, outv, osem):
            pix = (base + it) * CHUNK
            pltpu.async_copy(outv, out_hbm.at[pl.ds(pix, CHUNK)], osem)

        def drain_out(outv, osem):
            pltpu.make_async_copy(
                outv, out_hbm.at[pl.ds(0, CHUNK)], osem).wait()

        def blend(it, rowsv, outv):
            def pix_body(q, c2):
                qg = it * CHUNK + q
                w16 = w_v[qg // 4, :]
                lb = (qg % 4) * 4

                def splat(t):
                    iv = jnp.full((16,), lb + t, jnp.int32)
                    return lax.gather(
                        w16, iv[:, None], _GATHER_DNUMS, (1,),
                        mode=lax.GatherScatterMode.PROMISE_IN_BOUNDS)

                w0 = splat(0)
                w1 = splat(1)
                w2 = splat(2)
                w3 = splat(3)
                for cc in range(C // 16):
                    s = pl.ds(cc * 16, 16)
                    acc = (w0 * rowsv[4 * q, s] + w1 * rowsv[4 * q + 1, s]
                           + w2 * rowsv[4 * q + 2, s]
                           + w3 * rowsv[4 * q + 3, s])
                    outv[q, s] = acc
                return c2

            lax.fori_loop(0, CHUNK, pix_body, 0)

        fire(0, rows0, sem0)
        fire(1, rows1, sem1)

        def body(g, carry):
            it0 = 2 * g
            it1 = 2 * g + 1
            drain_rows(rows0, sem0)

            @pl.when(g > 0)
            def _():
                drain_out(out0, osem0)

            blend(it0, rows0, out0)

            @pl.when(it0 + 2 < CHUNKS_PER_W)
            def _():
                fire(it0 + 2, rows0, sem0)

            put(it0, out0, osem0)

            drain_rows(rows1, sem1)

            @pl.when(g > 0)
            def _():
                drain_out(out1, osem1)

            blend(it1, rows1, out1)

            @pl.when(it1 + 2 < CHUNKS_PER_W)
            def _():
                fire(it1 + 2, rows1, sem1)

            put(it1, out1, osem1)
            return carry

        lax.fori_loop(0, CHUNKS_PER_W // 2, body, 0)
        drain_out(out0, osem0)
        drain_out(out1, osem1)

    return sc_body


def kernel(bboxes, fm0, fm1, fm2, fm3, fm4):
    bb_t = bboxes.reshape(R, 4).T.reshape(4, 8, 128)
    idx_t, w_t = _index_weights(bb_t)
    idx = idx_t.reshape(K, R).T.reshape(-1)
    wts = w_t.reshape(K, R).T.reshape(-1, 16)
    table = jnp.concatenate(
        [fm.reshape(-1, C) for fm in (fm0, fm1, fm2, fm3, fm4)], axis=0)
    out = _sc_gather_blend()(idx, wts, table)
    return out.reshape(BATCH, NBOX, CROP, CROP, C)
